# baseline (speedup 1.0000x reference)
import functools

import jax
import jax.numpy as jnp
from jax import lax
from jax.experimental import pallas as pl
from jax.experimental.pallas import tpu as pltpu
from jax.experimental.pallas import tpu_sc as plsc

_B, _C, _H, _W = 16, 768, 32, 32
_HALF = 384
_HW = _H * _W
_NW = 32
_CPW = _C // _NW


def _pos_body(row_hbm, col_hbm, out_hbm, tab_v, chunk_v, sem):
    cid = lax.axis_index("c")
    sid = lax.axis_index("s")
    wid = sid * 2 + cid
    c0 = wid * _CPW
    is_row_half = wid < (_NW // 2)

    @pl.when(is_row_half)
    def _():
        pltpu.sync_copy(row_hbm.at[pl.ds(0, _H * _HALF)], tab_v)

        def body_c(cp, carry):
            base = cp * _HW
            c = c0 + cp
            for h in range(_H):
                idx = jnp.full((16,), h * _HALF, jnp.int32) + c
                v = plsc.load_gather(tab_v, [idx])
                chunk_v[pl.ds(base + h * _W, 16)] = v
                chunk_v[pl.ds(base + h * _W + 16, 16)] = v
            return carry

        lax.fori_loop(0, _CPW, body_c, 0)

    @pl.when(jnp.logical_not(is_row_half))
    def _():
        pltpu.sync_copy(col_hbm.at[pl.ds(0, _W * _HALF)], tab_v)

        def body_c(cp, carry):
            base = cp * _HW
            c = (c0 - _HALF) + cp
            i16 = lax.iota(jnp.int32, 16)
            vlo = plsc.load_gather(tab_v, [i16 * _HALF + c])
            vhi = plsc.load_gather(tab_v, [(i16 + 16) * _HALF + c])
            for h in range(_H):
                chunk_v[pl.ds(base + h * _W, 16)] = vlo
                chunk_v[pl.ds(base + h * _W + 16, 16)] = vhi
            return carry

        lax.fori_loop(0, _CPW, body_c, 0)

    copies = [
        pltpu.async_copy(chunk_v, out_hbm.at[b, pl.ds(c0 * _HW, _CPW * _HW)], sem)
        for b in range(_B)
    ]
    for c in copies:
        c.wait()


@jax.jit
def _pos_encode(row_embed, col_embed):
    mesh = plsc.VectorSubcoreMesh(core_axis_name="c", subcore_axis_name="s")
    run = functools.partial(
        pl.kernel,
        out_type=jax.ShapeDtypeStruct((_B, _C * _HW), jnp.float32),
        mesh=mesh,
        compiler_params=pltpu.CompilerParams(needs_layout_passes=False),
        scratch_types=[
            pltpu.VMEM((_H * _HALF,), jnp.float32),
            pltpu.VMEM((_CPW * _HW,), jnp.float32),
            pltpu.SemaphoreType.DMA,
        ],
    )(_pos_body)
    flat = run(row_embed.reshape(-1), col_embed.reshape(-1))
    return flat.reshape(_B, _C, _H, _W)


def kernel(feat, row_embed, col_embed):
    del feat
    return _pos_encode(row_embed, col_embed)
